# baseline (speedup 1.0000x reference)
import functools

import jax
import jax.numpy as jnp
from jax import lax
from jax.experimental import pallas as pl
from jax.experimental.pallas import tpu as pltpu
from jax.experimental.pallas import tpu_sc as plsc

N = 50000
E = 800000
LANES = 128
EROWS = 6400
EPAD = EROWS * LANES
NACC = 50048
DUMP = N
CH = 40
ZROWS = 184
NPAD = 51200
RB = 2048
GRID = NPAD // RB


def _fill2d(ref, rows, width, value):
    v = jnp.full((16,), value, dtype=jnp.float32)

    def body(i, _):
        for f in range(width // 16):
            ref[i, pl.ds(f * 16, 16)] = v
        return 0

    lax.fori_loop(0, rows, body, 0)


def _sc_mesh():
    return plsc.VectorSubcoreMesh(core_axis_name="c", subcore_axis_name="s")


_SC_PARAMS = pltpu.CompilerParams(use_tc_tiling_on_sc=False)


def _make_deg_kernel():
    @functools.partial(
        pl.kernel,
        out_type=jax.ShapeDtypeStruct((2, NACC, 16), jnp.float32),
        mesh=_sc_mesh(),
        compiler_params=_SC_PARAMS,
        scratch_types=[
            pltpu.VMEM((CH, LANES), jnp.int32),
            pltpu.VMEM((LANES, 16), jnp.float32),
            pltpu.VMEM((ZROWS, 16), jnp.float32),
            pltpu.VMEM_SHARED((NACC, 16), jnp.float32),
        ],
    )
    def k(dstp, out, didx, ones_v, zbuf, acc):
        c = lax.axis_index("c")
        s = lax.axis_index("s")
        _fill2d(ones_v, LANES, 16, 1.0)
        _fill2d(zbuf, ZROWS, 16, 0.0)
        zbase = s * (NACC // 16)
        for z in range(17):
            pltpu.sync_copy(zbuf, acc.at[pl.ds(zbase + z * ZROWS, ZROWS)])
        plsc.subcore_barrier()

        rows_per_tile = EROWS // 32
        row_base = (c * 16 + s) * rows_per_tile

        def outer(ob, _):
            rb = row_base + ob * CH
            pltpu.sync_copy(dstp.at[pl.ds(rb, CH)], didx)

            def inner(j, _):
                pltpu.sync_copy(ones_v, acc.at[didx.at[j]], add=True)
                return 0

            lax.fori_loop(0, CH, inner, 0)
            return 0

        lax.fori_loop(0, rows_per_tile // CH, outer, 0)
        plsc.subcore_barrier()
        nout = NACC // 16
        pltpu.sync_copy(acc.at[pl.ds(s * nout, nout)],
                        out.at[c, pl.ds(s * nout, nout)])

    return k


def _make_agg_kernel(f2, colsplit):
    n_in = 2 if colsplit else 1
    rows_per_tile = EROWS // 16 if colsplit else EROWS // 32

    def body(*refs):
        tables = refs[:n_in]
        srcp, dstp = refs[n_in], refs[n_in + 1]
        out = refs[n_in + 2]
        (sidx, didx, rows_a, rows_b, zbuf, acc,
         gsem_a, gsem_b, ssem_a, ssem_b) = refs[n_in + 3:]
        c = lax.axis_index("c")
        s = lax.axis_index("s")
        _fill2d(zbuf, ZROWS, f2, 0.0)
        zbase = s * (NACC // 16)
        for z in range(17):
            pltpu.sync_copy(zbuf, acc.at[pl.ds(zbase + z * ZROWS, ZROWS)])
        plsc.subcore_barrier()

        if colsplit:
            row_base = s * rows_per_tile
        else:
            row_base = (c * 16 + s) * rows_per_tile

        bufs = (rows_a, rows_b)
        gsems = (gsem_a, gsem_b)
        ssems = (ssem_a, ssem_b)

        def gather(j, k):
            if colsplit:
                @pl.when(c == 0)
                def _():
                    pltpu.async_copy(tables[0].at[sidx.at[j]], bufs[k],
                                     gsems[k])

                @pl.when(c == 1)
                def _():
                    pltpu.async_copy(tables[1].at[sidx.at[j]], bufs[k],
                                     gsems[k])
            else:
                pltpu.async_copy(tables[0].at[sidx.at[j]], bufs[k], gsems[k])

        def gwait(j, k):
            pltpu.make_async_copy(tables[0].at[sidx.at[j]], bufs[k],
                                  gsems[k]).wait()

        def scat(j, k):
            pltpu.async_copy(bufs[k], acc.at[didx.at[j]], ssems[k], add=True)

        def swait(j, k):
            pltpu.make_async_copy(bufs[k], acc.at[didx.at[j]],
                                  ssems[k]).wait()

        def outer(ob, _):
            rb = row_base + ob * CH
            pltpu.sync_copy(srcp.at[pl.ds(rb, CH)], sidx)
            pltpu.sync_copy(dstp.at[pl.ds(rb, CH)], didx)
            gather(0, 0)
            gather(1, 1)

            def inner2(j2, _):
                j0 = 2 * j2
                gwait(j0, 0)
                scat(j0, 0)
                gwait(j0 + 1, 1)
                scat(j0 + 1, 1)

                @pl.when(j2 + 1 < CH // 2)
                def _():
                    swait(j0, 0)
                    gather(j0 + 2, 0)
                    swait(j0 + 1, 1)
                    gather(j0 + 3, 1)
                return 0

            lax.fori_loop(0, CH // 2, inner2, 0)
            swait(CH - 2, 0)
            swait(CH - 1, 1)
            return 0

        lax.fori_loop(0, rows_per_tile // CH, outer, 0)
        plsc.subcore_barrier()
        nout = NACC // 16
        pltpu.sync_copy(acc.at[pl.ds(s * nout, nout)],
                        out.at[c, pl.ds(s * nout, nout)])

    return functools.partial(
        pl.kernel,
        out_type=jax.ShapeDtypeStruct((2, NACC, f2), jnp.float32),
        mesh=_sc_mesh(),
        compiler_params=_SC_PARAMS,
        scratch_types=[
            pltpu.VMEM((CH, LANES), jnp.int32),
            pltpu.VMEM((CH, LANES), jnp.int32),
            pltpu.VMEM((LANES, f2), jnp.float32),
            pltpu.VMEM((LANES, f2), jnp.float32),
            pltpu.VMEM((ZROWS, f2), jnp.float32),
            pltpu.VMEM_SHARED((NACC, f2), jnp.float32),
            pltpu.SemaphoreType.DMA,
            pltpu.SemaphoreType.DMA,
            pltpu.SemaphoreType.DMA,
            pltpu.SemaphoreType.DMA,
        ],
    )(body)


_deg_call = _make_deg_kernel()
_agg64 = _make_agg_kernel(32, colsplit=True)
_agg32 = _make_agg_kernel(32, colsplit=False)
_agg16 = _make_agg_kernel(16, colsplit=False)


def _full(shape):
    return pl.BlockSpec(shape, lambda i: (0,) * len(shape))


def _rows(shape):
    nd = len(shape)
    if nd == 2:
        return pl.BlockSpec(shape, lambda i: (i, 0))
    return pl.BlockSpec(shape, lambda i: (0, i, 0))


def _tc1_body(x_ref, wint_ref, bin_ref, g1_ref, b1_ref, wg1t_ref,
              h_ref, hl1_ref):
    xb = x_ref[...]
    h0 = jnp.maximum(
        jnp.dot(xb, wint_ref[...], preferred_element_type=jnp.float32)
        + bin_ref[...], 0.0)
    m = jnp.mean(h0, axis=-1, keepdims=True)
    v = jnp.mean((h0 - m) ** 2, axis=-1, keepdims=True)
    hb = (h0 - m) / jnp.sqrt(v + 1e-5) * g1_ref[...] + b1_ref[...]
    h_ref[...] = hb
    hl1_ref[...] = jnp.dot(hb, wg1t_ref[...],
                           preferred_element_type=jnp.float32)


def _tc1(xp, wint, bin_, g1, b1, wg1t):
    return pl.pallas_call(
        _tc1_body,
        grid=(GRID,),
        in_specs=[
            _rows((RB, 192)),
            _full((192, 64)),
            _full((1, 64)),
            _full((1, 64)),
            _full((1, 64)),
            _full((64, 64)),
        ],
        out_specs=[_rows((RB, 64)), _rows((RB, 64))],
        out_shape=[
            jax.ShapeDtypeStruct((NPAD, 64), jnp.float32),
            jax.ShapeDtypeStruct((NPAD, 64), jnp.float32),
        ],
    )(xp, wint, bin_, g1, b1, wg1t)


def _tc2_body(h_ref, a1_ref, hl1_ref, dinv_ref, bg1_ref, wg2t_ref, hl2_ref):
    dinv = dinv_ref[...]
    hs1 = hl1_ref[...] * dinv
    left = a1_ref[0] + hs1[:, :32]
    right = a1_ref[1] + hs1[:, 32:]
    agg = jnp.concatenate([left, right], axis=1)
    t = jnp.maximum(agg * dinv + bg1_ref[...], 0.0)
    h1 = t + h_ref[...]
    hl2_ref[...] = jnp.dot(h1, wg2t_ref[...],
                           preferred_element_type=jnp.float32)


def _tc2(h, a1, hl1, dinv, bg1, wg2t):
    return pl.pallas_call(
        _tc2_body,
        grid=(GRID,),
        in_specs=[
            _rows((RB, 64)),
            _rows((2, RB, 32)),
            _rows((RB, 64)),
            _rows((RB, 1)),
            _full((1, 64)),
            _full((64, 32)),
        ],
        out_specs=[_rows((RB, 32))],
        out_shape=[jax.ShapeDtypeStruct((NPAD, 32), jnp.float32)],
    )(h, a1, hl1, dinv, bg1, wg2t)[0]


def _tc3_body(a2_ref, hl2_ref, dinv_ref, bg2_ref, g2_ref, b2_ref, wg3t_ref,
              hl3_ref):
    dinv = dinv_ref[...]
    agg = a2_ref[0] + a2_ref[1] + hl2_ref[...] * dinv
    t = jnp.maximum(agg * dinv + bg2_ref[...], 0.0)
    m = jnp.mean(t, axis=-1, keepdims=True)
    v = jnp.mean((t - m) ** 2, axis=-1, keepdims=True)
    h2 = (t - m) / jnp.sqrt(v + 1e-5) * g2_ref[...] + b2_ref[...]
    hl3_ref[...] = jnp.dot(h2, wg3t_ref[...],
                           preferred_element_type=jnp.float32)


def _tc3(a2, hl2, dinv, bg2, g2, b2, wg3t):
    return pl.pallas_call(
        _tc3_body,
        grid=(GRID,),
        in_specs=[
            _rows((2, RB, 32)),
            _rows((RB, 32)),
            _rows((RB, 1)),
            _full((1, 32)),
            _full((1, 32)),
            _full((1, 32)),
            _full((32, 16)),
        ],
        out_specs=[_rows((RB, 16))],
        out_shape=[jax.ShapeDtypeStruct((NPAD, 16), jnp.float32)],
    )(a2, hl2, dinv, bg2, g2, b2, wg3t)[0]


def _tc4_body(a3_ref, hl3_ref, dinv_ref, bg3_ref, wvt_ref, bv_ref, wot_ref,
              bo_ref, wc1t_ref, bc1_ref, wc2t_ref, bc2_ref, wc3t_ref,
              bc3_ref, out_ref):
    dinv = dinv_ref[...]
    agg = a3_ref[0] + a3_ref[1] + hl3_ref[...] * dinv
    h3 = jnp.maximum(agg * dinv + bg3_ref[...], 0.0)
    vv = jnp.dot(h3, wvt_ref[...], preferred_element_type=jnp.float32) \
        + bv_ref[...]
    att = jnp.dot(vv, wot_ref[...], preferred_element_type=jnp.float32) \
        + bo_ref[...]
    p = jnp.maximum(
        jnp.dot(att, wc1t_ref[...], preferred_element_type=jnp.float32)
        + bc1_ref[...], 0.0)
    p = jnp.maximum(
        jnp.dot(p, wc2t_ref[...], preferred_element_type=jnp.float32)
        + bc2_ref[...], 0.0)
    out_ref[...] = jnp.dot(p, wc3t_ref[...],
                           preferred_element_type=jnp.float32) + bc3_ref[...]


def _tc4(a3, hl3, dinv, bg3, wvt, bv, wot, bo, wc1t, bc1, wc2t, bc2, wc3t,
         bc3):
    return pl.pallas_call(
        _tc4_body,
        grid=(GRID,),
        in_specs=[
            _rows((2, RB, 16)),
            _rows((RB, 16)),
            _rows((RB, 1)),
            _full((1, 16)),
            _full((16, 16)),
            _full((1, 16)),
            _full((16, 16)),
            _full((1, 16)),
            _full((16, 8)),
            _full((1, 8)),
            _full((8, 32)),
            _full((1, 32)),
            _full((32, 1)),
            _full((1, 1)),
        ],
        out_specs=[_rows((RB, 1))],
        out_shape=[jax.ShapeDtypeStruct((NPAD, 1), jnp.float32)],
    )(a3, hl3, dinv, bg3, wvt, bv, wot, bo, wc1t, bc1, wc2t, bc2, wc3t,
      bc3)[0]


def kernel(x, edge_index, W_in, b_in, ln1_g, ln1_b, Wg1, bg1, Wg2, bg2,
           ln2_g, ln2_b, Wg3, bg3, Wqkv, bqkv, Wo, bo, Wc1, bc1, Wc2, bc2,
           Wc3, bc3):
    src = edge_index[0]
    dst = edge_index[1]
    srcp = jnp.pad(src, (0, EPAD - E)).reshape(EROWS, LANES)
    dstp = jnp.pad(dst, (0, EPAD - E),
                   constant_values=DUMP).reshape(EROWS, LANES)

    xp = jnp.pad(x, ((0, NPAD - N), (0, 192 - x.shape[1])))
    wint = jnp.pad(W_in.T, ((0, 192 - W_in.shape[1]), (0, 0)))

    degp = _deg_call(dstp)
    deg = degp[0, :, 0] + degp[1, :, 0] + 1.0
    dinv1 = lax.rsqrt(deg)
    dinv = jnp.pad(dinv1, (0, NPAD - NACC),
                   constant_values=1.0).reshape(NPAD, 1)

    h, hl1 = _tc1(xp, wint, b_in.reshape(1, 64), ln1_g.reshape(1, 64),
                  ln1_b.reshape(1, 64), Wg1.T)

    hs1 = hl1 * dinv
    a1 = _agg64(hs1[:, :32], hs1[:, 32:], srcp, dstp)

    hl2 = _tc2(h, a1, hl1, dinv, bg1.reshape(1, 64), Wg2.T)

    a2 = _agg32(hl2 * dinv, srcp, dstp)

    hl3 = _tc3(a2, hl2, dinv, bg2.reshape(1, 32), ln2_g.reshape(1, 32),
               ln2_b.reshape(1, 32), Wg3.T)

    a3 = _agg16(hl3 * dinv, srcp, dstp)

    wv = Wqkv[32:48]
    bv = bqkv[32:48]
    out = _tc4(a3, hl3, dinv, bg3.reshape(1, 16), wv.T, bv.reshape(1, 16),
               Wo.T, bo.reshape(1, 16), Wc1.T, bc1.reshape(1, 8), Wc2.T,
               bc2.reshape(1, 32), Wc3.T, bc3.reshape(1, 1))
    return out[:N, 0]


# speedup vs baseline: 1.3859x; 1.3859x over previous
import functools

import jax
import jax.numpy as jnp
from jax import lax
from jax.experimental import pallas as pl
from jax.experimental.pallas import tpu as pltpu
from jax.experimental.pallas import tpu_sc as plsc

N = 50000
E = 800000
LANES = 128
EROWS = E // LANES
NACC = 50048
CH = 40
ZROWS = 184
NPAD = 51200
RB = 2048
GRID = NPAD // RB


def _fill2d(ref, rows, width, value):
    v = jnp.full((16,), value, dtype=jnp.float32)

    def body(i, _):
        for f in range(width // 16):
            ref[i, pl.ds(f * 16, 16)] = v
        return 0

    lax.fori_loop(0, rows, body, 0)


def _sc_mesh():
    return plsc.VectorSubcoreMesh(core_axis_name="c", subcore_axis_name="s")


_SC_PARAMS = pltpu.CompilerParams(use_tc_tiling_on_sc=False)


def _make_deg_kernel():
    @functools.partial(
        pl.kernel,
        out_type=jax.ShapeDtypeStruct((2, NACC, 16), jnp.float32),
        mesh=_sc_mesh(),
        compiler_params=_SC_PARAMS,
        scratch_types=[
            pltpu.VMEM((CH, 2, LANES), jnp.int32),
            pltpu.VMEM((LANES, 16), jnp.float32),
            pltpu.VMEM((ZROWS, 16), jnp.float32),
            pltpu.VMEM_SHARED((NACC, 16), jnp.float32),
        ],
    )
    def k(e3, out, sd, ones_v, zbuf, acc):
        c = lax.axis_index("c")
        s = lax.axis_index("s")
        _fill2d(ones_v, LANES, 16, 1.0)
        _fill2d(zbuf, ZROWS, 16, 0.0)
        zbase = s * (NACC // 16)
        for z in range(17):
            pltpu.sync_copy(zbuf, acc.at[pl.ds(zbase + z * ZROWS, ZROWS)])
        plsc.subcore_barrier()

        wid = c * 16 + s
        row_base = wid * 195 + jnp.minimum(wid, 10)

        def run_rows(rb, nrows):
            pltpu.sync_copy(e3.at[pl.ds(rb, nrows)], sd.at[pl.ds(0, nrows)])

            def inner(j, _):
                pltpu.sync_copy(ones_v, acc.at[sd.at[j, 1]], add=True)
                return 0

            lax.fori_loop(0, nrows, inner, 0)

        def outer(ob, _):
            run_rows(row_base + ob * CH, CH)
            return 0

        lax.fori_loop(0, 4, outer, 0)
        run_rows(row_base + 4 * CH, 35)

        @pl.when(wid < 10)
        def _():
            run_rows(row_base + 195, 1)

        plsc.subcore_barrier()
        nout = NACC // 16
        pltpu.sync_copy(acc.at[pl.ds(s * nout, nout)],
                        out.at[c, pl.ds(s * nout, nout)])

    return k


def _make_agg_kernel(f2, colsplit):
    n_in = 2 if colsplit else 1
    base_rows, nfull, tail = (390, 9, 30) if colsplit else (195, 4, 35)

    def body(*refs):
        tables = refs[:n_in]
        e3 = refs[n_in]
        out = refs[n_in + 1]
        (sd, rows_a, rows_b, zbuf, acc,
         gsem_a, gsem_b, ssem_a, ssem_b) = refs[n_in + 2:]
        c = lax.axis_index("c")
        s = lax.axis_index("s")
        _fill2d(zbuf, ZROWS, f2, 0.0)
        zbase = s * (NACC // 16)
        for z in range(17):
            pltpu.sync_copy(zbuf, acc.at[pl.ds(zbase + z * ZROWS, ZROWS)])
        plsc.subcore_barrier()

        if colsplit:
            row_base = s * base_rows + jnp.minimum(s, 10)
            extra = s < 10
        else:
            wid = c * 16 + s
            row_base = wid * base_rows + jnp.minimum(wid, 10)
            extra = wid < 10

        bufs = (rows_a, rows_b)
        gsems = (gsem_a, gsem_b)
        ssems = (ssem_a, ssem_b)

        def gather(j, k):
            if colsplit:
                @pl.when(c == 0)
                def _():
                    pltpu.async_copy(tables[0].at[sd.at[j, 0]], bufs[k],
                                     gsems[k])

                @pl.when(c == 1)
                def _():
                    pltpu.async_copy(tables[1].at[sd.at[j, 0]], bufs[k],
                                     gsems[k])
            else:
                pltpu.async_copy(tables[0].at[sd.at[j, 0]], bufs[k],
                                 gsems[k])

        def gwait(j, k):
            pltpu.make_async_copy(tables[0].at[sd.at[j, 0]], bufs[k],
                                  gsems[k]).wait()

        def scat(j, k):
            pltpu.async_copy(bufs[k], acc.at[sd.at[j, 1]], ssems[k],
                             add=True)

        def swait(j, k):
            pltpu.make_async_copy(bufs[k], acc.at[sd.at[j, 1]],
                                  ssems[k]).wait()

        def run_rows(rb, nrows):
            pltpu.sync_copy(e3.at[pl.ds(rb, nrows)], sd.at[pl.ds(0, nrows)])
            npair = nrows // 2
            gather(0, 0)
            if nrows > 1:
                gather(1, 1)

            def inner2(j2, _):
                j0 = 2 * j2
                gwait(j0, 0)
                scat(j0, 0)
                gwait(j0 + 1, 1)
                scat(j0 + 1, 1)

                @pl.when(j2 + 1 < npair)
                def _():
                    swait(j0, 0)
                    gather(j0 + 2, 0)
                    swait(j0 + 1, 1)
                    gather(j0 + 3, 1)
                return 0

            if npair:
                lax.fori_loop(0, npair, inner2, 0)
                swait(2 * npair - 2, 0)
                swait(2 * npair - 1, 1)
            if nrows % 2:
                j = nrows - 1
                if nrows > 1:
                    gather(j, 0)
                gwait(j, 0)
                pltpu.sync_copy(bufs[0], acc.at[sd.at[j, 1]], add=True)

        def outer(ob, _):
            run_rows(row_base + ob * CH, CH)
            return 0

        lax.fori_loop(0, nfull, outer, 0)
        run_rows(row_base + nfull * CH, tail)

        @pl.when(extra)
        def _():
            run_rows(row_base + base_rows, 1)

        plsc.subcore_barrier()
        nout = NACC // 16
        pltpu.sync_copy(acc.at[pl.ds(s * nout, nout)],
                        out.at[c, pl.ds(s * nout, nout)])

    return functools.partial(
        pl.kernel,
        out_type=jax.ShapeDtypeStruct((2, NACC, f2), jnp.float32),
        mesh=_sc_mesh(),
        compiler_params=_SC_PARAMS,
        scratch_types=[
            pltpu.VMEM((CH, 2, LANES), jnp.int32),
            pltpu.VMEM((LANES, f2), jnp.float32),
            pltpu.VMEM((LANES, f2), jnp.float32),
            pltpu.VMEM((ZROWS, f2), jnp.float32),
            pltpu.VMEM_SHARED((NACC, f2), jnp.float32),
            pltpu.SemaphoreType.DMA,
            pltpu.SemaphoreType.DMA,
            pltpu.SemaphoreType.DMA,
            pltpu.SemaphoreType.DMA,
        ],
    )(body)


_deg_call = _make_deg_kernel()
_agg64 = _make_agg_kernel(32, colsplit=True)
_agg32 = _make_agg_kernel(32, colsplit=False)
_agg16 = _make_agg_kernel(16, colsplit=False)


def _full(shape):
    return pl.BlockSpec(shape, lambda i: (0,) * len(shape))


def _rows(shape):
    nd = len(shape)
    if nd == 2:
        return pl.BlockSpec(shape, lambda i: (i, 0))
    return pl.BlockSpec(shape, lambda i: (0, i, 0))


def _tc1_body(x_ref, wint_ref, bin_ref, g1_ref, b1_ref, wg1t_ref,
              h_ref, hl1_ref):
    xb = x_ref[...]
    h0 = jnp.maximum(
        jnp.dot(xb, wint_ref[...], preferred_element_type=jnp.float32)
        + bin_ref[...], 0.0)
    m = jnp.mean(h0, axis=-1, keepdims=True)
    v = jnp.mean((h0 - m) ** 2, axis=-1, keepdims=True)
    hb = (h0 - m) / jnp.sqrt(v + 1e-5) * g1_ref[...] + b1_ref[...]
    h_ref[...] = hb
    hl1_ref[...] = jnp.dot(hb, wg1t_ref[...],
                           preferred_element_type=jnp.float32)


def _tc1(xp, wint, bin_, g1, b1, wg1t):
    return pl.pallas_call(
        _tc1_body,
        grid=(GRID,),
        in_specs=[
            _rows((RB, 192)),
            _full((192, 64)),
            _full((1, 64)),
            _full((1, 64)),
            _full((1, 64)),
            _full((64, 64)),
        ],
        out_specs=[_rows((RB, 64)), _rows((RB, 64))],
        out_shape=[
            jax.ShapeDtypeStruct((NPAD, 64), jnp.float32),
            jax.ShapeDtypeStruct((NPAD, 64), jnp.float32),
        ],
    )(xp, wint, bin_, g1, b1, wg1t)


def _tc2_body(h_ref, a1_ref, hl1_ref, dinv_ref, bg1_ref, wg2t_ref, hl2_ref):
    dinv = dinv_ref[...]
    hs1 = hl1_ref[...] * dinv
    left = a1_ref[0] + hs1[:, :32]
    right = a1_ref[1] + hs1[:, 32:]
    agg = jnp.concatenate([left, right], axis=1)
    t = jnp.maximum(agg * dinv + bg1_ref[...], 0.0)
    h1 = t + h_ref[...]
    hl2_ref[...] = jnp.dot(h1, wg2t_ref[...],
                           preferred_element_type=jnp.float32)


def _tc2(h, a1, hl1, dinv, bg1, wg2t):
    return pl.pallas_call(
        _tc2_body,
        grid=(GRID,),
        in_specs=[
            _rows((RB, 64)),
            _rows((2, RB, 32)),
            _rows((RB, 64)),
            _rows((RB, 1)),
            _full((1, 64)),
            _full((64, 32)),
        ],
        out_specs=[_rows((RB, 32))],
        out_shape=[jax.ShapeDtypeStruct((NPAD, 32), jnp.float32)],
    )(h, a1, hl1, dinv, bg1, wg2t)[0]


def _tc3_body(a2_ref, hl2_ref, dinv_ref, bg2_ref, g2_ref, b2_ref, wg3t_ref,
              hl3_ref):
    dinv = dinv_ref[...]
    agg = a2_ref[0] + a2_ref[1] + hl2_ref[...] * dinv
    t = jnp.maximum(agg * dinv + bg2_ref[...], 0.0)
    m = jnp.mean(t, axis=-1, keepdims=True)
    v = jnp.mean((t - m) ** 2, axis=-1, keepdims=True)
    h2 = (t - m) / jnp.sqrt(v + 1e-5) * g2_ref[...] + b2_ref[...]
    hl3_ref[...] = jnp.dot(h2, wg3t_ref[...],
                           preferred_element_type=jnp.float32)


def _tc3(a2, hl2, dinv, bg2, g2, b2, wg3t):
    return pl.pallas_call(
        _tc3_body,
        grid=(GRID,),
        in_specs=[
            _rows((2, RB, 32)),
            _rows((RB, 32)),
            _rows((RB, 1)),
            _full((1, 32)),
            _full((1, 32)),
            _full((1, 32)),
            _full((32, 16)),
        ],
        out_specs=[_rows((RB, 16))],
        out_shape=[jax.ShapeDtypeStruct((NPAD, 16), jnp.float32)],
    )(a2, hl2, dinv, bg2, g2, b2, wg3t)[0]


def _tc4_body(a3_ref, hl3_ref, dinv_ref, bg3_ref, wvt_ref, bv_ref, wot_ref,
              bo_ref, wc1t_ref, bc1_ref, wc2t_ref, bc2_ref, wc3t_ref,
              bc3_ref, out_ref):
    dinv = dinv_ref[...]
    agg = a3_ref[0] + a3_ref[1] + hl3_ref[...] * dinv
    h3 = jnp.maximum(agg * dinv + bg3_ref[...], 0.0)
    vv = jnp.dot(h3, wvt_ref[...], preferred_element_type=jnp.float32) \
        + bv_ref[...]
    att = jnp.dot(vv, wot_ref[...], preferred_element_type=jnp.float32) \
        + bo_ref[...]
    p = jnp.maximum(
        jnp.dot(att, wc1t_ref[...], preferred_element_type=jnp.float32)
        + bc1_ref[...], 0.0)
    p = jnp.maximum(
        jnp.dot(p, wc2t_ref[...], preferred_element_type=jnp.float32)
        + bc2_ref[...], 0.0)
    out_ref[...] = jnp.dot(p, wc3t_ref[...],
                           preferred_element_type=jnp.float32) + bc3_ref[...]


def _tc4(a3, hl3, dinv, bg3, wvt, bv, wot, bo, wc1t, bc1, wc2t, bc2, wc3t,
         bc3):
    return pl.pallas_call(
        _tc4_body,
        grid=(GRID,),
        in_specs=[
            _rows((2, RB, 16)),
            _rows((RB, 16)),
            _rows((RB, 1)),
            _full((1, 16)),
            _full((16, 16)),
            _full((1, 16)),
            _full((16, 16)),
            _full((1, 16)),
            _full((16, 8)),
            _full((1, 8)),
            _full((8, 32)),
            _full((1, 32)),
            _full((32, 1)),
            _full((1, 1)),
        ],
        out_specs=[_rows((RB, 1))],
        out_shape=[jax.ShapeDtypeStruct((NPAD, 1), jnp.float32)],
    )(a3, hl3, dinv, bg3, wvt, bv, wot, bo, wc1t, bc1, wc2t, bc2, wc3t,
      bc3)[0]


def kernel(x, edge_index, W_in, b_in, ln1_g, ln1_b, Wg1, bg1, Wg2, bg2,
           ln2_g, ln2_b, Wg3, bg3, Wqkv, bqkv, Wo, bo, Wc1, bc1, Wc2, bc2,
           Wc3, bc3):
    e3 = jnp.transpose(edge_index.reshape(2, EROWS, LANES), (1, 0, 2))

    xp = jnp.pad(x, ((0, NPAD - N), (0, 192 - x.shape[1])))
    wint = jnp.pad(W_in.T, ((0, 192 - W_in.shape[1]), (0, 0)))

    degp = _deg_call(e3)
    deg = degp[0, :, 0] + degp[1, :, 0] + 1.0
    dinv1 = lax.rsqrt(deg)
    dinv = jnp.pad(dinv1, (0, NPAD - NACC),
                   constant_values=1.0).reshape(NPAD, 1)

    h, hl1 = _tc1(xp, wint, b_in.reshape(1, 64), ln1_g.reshape(1, 64),
                  ln1_b.reshape(1, 64), Wg1.T)

    hs1 = hl1 * dinv
    a1 = _agg64(hs1[:, :32], hs1[:, 32:], e3)

    hl2 = _tc2(h, a1, hl1, dinv, bg1.reshape(1, 64), Wg2.T)

    a2 = _agg32(hl2 * dinv, e3)

    hl3 = _tc3(a2, hl2, dinv, bg2.reshape(1, 32), ln2_g.reshape(1, 32),
               ln2_b.reshape(1, 32), Wg3.T)

    a3 = _agg16(hl3 * dinv, e3)

    wv = Wqkv[32:48]
    bv = bqkv[32:48]
    out = _tc4(a3, hl3, dinv, bg3.reshape(1, 16), wv.T, bv.reshape(1, 16),
               Wo.T, bo.reshape(1, 16), Wc1.T, bc1.reshape(1, 8), Wc2.T,
               bc2.reshape(1, 32), Wc3.T, bc3.reshape(1, 1))
    return out[:N, 0]


# speedup vs baseline: 1.7038x; 1.2294x over previous
import functools

import jax
import jax.numpy as jnp
from jax import lax
from jax.experimental import pallas as pl
from jax.experimental.pallas import tpu as pltpu
from jax.experimental.pallas import tpu_sc as plsc

N = 50000
E = 800000
LANES = 128
EROWS = E // LANES
NACC = 50048
CH = 40
ZROWS = 184
NPAD = 51200
RB = 2048
GRID = NPAD // RB


def _fill2d(ref, rows, width, value):
    v = jnp.full((16,), value, dtype=jnp.float32)

    def body(i, _):
        for f in range(width // 16):
            ref[i, pl.ds(f * 16, 16)] = v
        return 0

    lax.fori_loop(0, rows, body, 0)


def _sc_mesh():
    return plsc.VectorSubcoreMesh(core_axis_name="c", subcore_axis_name="s")


_SC_PARAMS = pltpu.CompilerParams(use_tc_tiling_on_sc=False)


def _make_deg_kernel():
    @functools.partial(
        pl.kernel,
        out_type=jax.ShapeDtypeStruct((2, NACC, 16), jnp.float32),
        mesh=_sc_mesh(),
        compiler_params=_SC_PARAMS,
        scratch_types=[
            pltpu.VMEM((CH, 2, LANES), jnp.int32),
            pltpu.VMEM((LANES, 16), jnp.float32),
            pltpu.VMEM((ZROWS, 16), jnp.float32),
            pltpu.VMEM_SHARED((NACC, 16), jnp.float32),
        ],
    )
    def k(e3, out, sd, ones_v, zbuf, acc):
        c = lax.axis_index("c")
        s = lax.axis_index("s")
        _fill2d(ones_v, LANES, 16, 1.0)
        _fill2d(zbuf, ZROWS, 16, 0.0)
        zbase = s * (NACC // 16)
        for z in range(17):
            pltpu.sync_copy(zbuf, acc.at[pl.ds(zbase + z * ZROWS, ZROWS)])
        plsc.subcore_barrier()

        wid = c * 16 + s
        row_base = wid * 195 + jnp.minimum(wid, 10)

        def run_rows(rb, nrows):
            pltpu.sync_copy(e3.at[pl.ds(rb, nrows)], sd.at[pl.ds(0, nrows)])

            def inner(j, _):
                pltpu.sync_copy(ones_v, acc.at[sd.at[j, 1]], add=True)
                return 0

            lax.fori_loop(0, nrows, inner, 0)

        def outer(ob, _):
            run_rows(row_base + ob * CH, CH)
            return 0

        lax.fori_loop(0, 4, outer, 0)
        run_rows(row_base + 4 * CH, 35)

        @pl.when(wid < 10)
        def _():
            run_rows(row_base + 195, 1)

        plsc.subcore_barrier()
        nout = NACC // 16
        pltpu.sync_copy(acc.at[pl.ds(s * nout, nout)],
                        out.at[c, pl.ds(s * nout, nout)])

    return k


def _make_agg_kernel(f2, colsplit):
    n_in = 2 if colsplit else 1
    base_rows, nfull, tail = (390, 9, 30) if colsplit else (195, 4, 35)

    def body(*refs):
        tables = refs[:n_in]
        e3 = refs[n_in]
        out = refs[n_in + 1]
        (sd, rows_a, rows_b, zbuf, acc,
         gsem_a, gsem_b, ssem_a, ssem_b) = refs[n_in + 2:]
        c = lax.axis_index("c")
        s = lax.axis_index("s")
        _fill2d(zbuf, ZROWS, f2, 0.0)
        zbase = s * (NACC // 16)
        for z in range(17):
            pltpu.sync_copy(zbuf, acc.at[pl.ds(zbase + z * ZROWS, ZROWS)])
        plsc.subcore_barrier()

        if colsplit:
            row_base = s * base_rows + jnp.minimum(s, 10)
            extra = s < 10
        else:
            wid = c * 16 + s
            row_base = wid * base_rows + jnp.minimum(wid, 10)
            extra = wid < 10

        bufs = (rows_a, rows_b)
        gsems = (gsem_a, gsem_b)
        ssems = (ssem_a, ssem_b)

        def gather(j, k):
            if colsplit:
                @pl.when(c == 0)
                def _():
                    pltpu.async_copy(tables[0].at[sd.at[j, 0]], bufs[k],
                                     gsems[k])

                @pl.when(c == 1)
                def _():
                    pltpu.async_copy(tables[1].at[sd.at[j, 0]], bufs[k],
                                     gsems[k])
            else:
                pltpu.async_copy(tables[0].at[sd.at[j, 0]], bufs[k],
                                 gsems[k])

        def gwait(j, k):
            pltpu.make_async_copy(tables[0].at[sd.at[j, 0]], bufs[k],
                                  gsems[k]).wait()

        def scat(j, k):
            pltpu.async_copy(bufs[k], acc.at[sd.at[j, 1]], ssems[k],
                             add=True)

        def swait(j, k):
            pltpu.make_async_copy(bufs[k], acc.at[sd.at[j, 1]],
                                  ssems[k]).wait()

        def run_rows(rb, nrows):
            pltpu.sync_copy(e3.at[pl.ds(rb, nrows)], sd.at[pl.ds(0, nrows)])
            npair = nrows // 2
            gather(0, 0)
            if nrows > 1:
                gather(1, 1)

            def inner2(j2, _):
                j0 = 2 * j2
                gwait(j0, 0)
                scat(j0, 0)
                gwait(j0 + 1, 1)
                scat(j0 + 1, 1)

                @pl.when(j2 + 1 < npair)
                def _():
                    swait(j0, 0)
                    gather(j0 + 2, 0)
                    swait(j0 + 1, 1)
                    gather(j0 + 3, 1)
                return 0

            if npair:
                lax.fori_loop(0, npair, inner2, 0)
                swait(2 * npair - 2, 0)
                swait(2 * npair - 1, 1)
            if nrows % 2:
                j = nrows - 1
                if nrows > 1:
                    gather(j, 0)
                gwait(j, 0)
                pltpu.sync_copy(bufs[0], acc.at[sd.at[j, 1]], add=True)

        def outer(ob, _):
            run_rows(row_base + ob * CH, CH)
            return 0

        lax.fori_loop(0, nfull, outer, 0)
        run_rows(row_base + nfull * CH, tail)

        @pl.when(extra)
        def _():
            run_rows(row_base + base_rows, 1)

        plsc.subcore_barrier()
        nout = NACC // 16
        pltpu.sync_copy(acc.at[pl.ds(s * nout, nout)],
                        out.at[c, pl.ds(s * nout, nout)])

    return functools.partial(
        pl.kernel,
        out_type=jax.ShapeDtypeStruct((2, NACC, f2), jnp.float32),
        mesh=_sc_mesh(),
        compiler_params=_SC_PARAMS,
        scratch_types=[
            pltpu.VMEM((CH, 2, LANES), jnp.int32),
            pltpu.VMEM((LANES, f2), jnp.float32),
            pltpu.VMEM((LANES, f2), jnp.float32),
            pltpu.VMEM((ZROWS, f2), jnp.float32),
            pltpu.VMEM_SHARED((NACC, f2), jnp.float32),
            pltpu.SemaphoreType.DMA,
            pltpu.SemaphoreType.DMA,
            pltpu.SemaphoreType.DMA,
            pltpu.SemaphoreType.DMA,
        ],
    )(body)


_deg_call = _make_deg_kernel()
_agg64 = _make_agg_kernel(32, colsplit=True)
_agg32 = _make_agg_kernel(32, colsplit=False)
_agg16 = _make_agg_kernel(16, colsplit=False)


def _full(shape):
    return pl.BlockSpec(shape, lambda i: (0,) * len(shape))


def _rows(shape):
    nd = len(shape)
    if nd == 2:
        return pl.BlockSpec(shape, lambda i: (i, 0))
    return pl.BlockSpec(shape, lambda i: (0, i, 0))


def _tc1_body(x_ref, wint_ref, bin_ref, g1_ref, b1_ref, wg1t_ref,
              h_ref, hl1_ref):
    xb = x_ref[...]
    h0 = jnp.maximum(
        lax.dot_general(xb, wint_ref[...], (((0,), (0,)), ((), ())),
                        preferred_element_type=jnp.float32)
        + bin_ref[...], 0.0)
    m = jnp.mean(h0, axis=-1, keepdims=True)
    v = jnp.mean((h0 - m) ** 2, axis=-1, keepdims=True)
    hb = (h0 - m) / jnp.sqrt(v + 1e-5) * g1_ref[...] + b1_ref[...]
    h_ref[...] = hb
    hl1_ref[...] = jnp.dot(hb, wg1t_ref[...],
                           preferred_element_type=jnp.float32)


def _tc1(xp, wint, bin_, g1, b1, wg1t):
    return pl.pallas_call(
        _tc1_body,
        grid=(GRID,),
        in_specs=[
            pl.BlockSpec((164, RB), lambda i: (0, i)),
            _full((164, 64)),
            _full((1, 64)),
            _full((1, 64)),
            _full((1, 64)),
            _full((64, 64)),
        ],
        out_specs=[_rows((RB, 64)), _rows((RB, 64))],
        out_shape=[
            jax.ShapeDtypeStruct((NPAD, 64), jnp.float32),
            jax.ShapeDtypeStruct((NPAD, 64), jnp.float32),
        ],
    )(xp, wint, bin_, g1, b1, wg1t)


def _tc2_body(h_ref, a1_ref, hl1_ref, dinv_ref, bg1_ref, wg2t_ref, hl2_ref):
    dinv = dinv_ref[...]
    hs1 = hl1_ref[...] * dinv
    left = a1_ref[0] + hs1[:, :32]
    right = a1_ref[1] + hs1[:, 32:]
    agg = jnp.concatenate([left, right], axis=1)
    t = jnp.maximum(agg * dinv + bg1_ref[...], 0.0)
    h1 = t + h_ref[...]
    hl2_ref[...] = jnp.dot(h1, wg2t_ref[...],
                           preferred_element_type=jnp.float32)


def _tc2(h, a1, hl1, dinv, bg1, wg2t):
    return pl.pallas_call(
        _tc2_body,
        grid=(GRID,),
        in_specs=[
            _rows((RB, 64)),
            _rows((2, RB, 32)),
            _rows((RB, 64)),
            _rows((RB, 1)),
            _full((1, 64)),
            _full((64, 32)),
        ],
        out_specs=[_rows((RB, 32))],
        out_shape=[jax.ShapeDtypeStruct((NPAD, 32), jnp.float32)],
    )(h, a1, hl1, dinv, bg1, wg2t)[0]


def _tc3_body(a2_ref, hl2_ref, dinv_ref, bg2_ref, g2_ref, b2_ref, wg3t_ref,
              hl3_ref):
    dinv = dinv_ref[...]
    agg = a2_ref[0] + a2_ref[1] + hl2_ref[...] * dinv
    t = jnp.maximum(agg * dinv + bg2_ref[...], 0.0)
    m = jnp.mean(t, axis=-1, keepdims=True)
    v = jnp.mean((t - m) ** 2, axis=-1, keepdims=True)
    h2 = (t - m) / jnp.sqrt(v + 1e-5) * g2_ref[...] + b2_ref[...]
    hl3_ref[...] = jnp.dot(h2, wg3t_ref[...],
                           preferred_element_type=jnp.float32)


def _tc3(a2, hl2, dinv, bg2, g2, b2, wg3t):
    return pl.pallas_call(
        _tc3_body,
        grid=(GRID,),
        in_specs=[
            _rows((2, RB, 32)),
            _rows((RB, 32)),
            _rows((RB, 1)),
            _full((1, 32)),
            _full((1, 32)),
            _full((1, 32)),
            _full((32, 16)),
        ],
        out_specs=[_rows((RB, 16))],
        out_shape=[jax.ShapeDtypeStruct((NPAD, 16), jnp.float32)],
    )(a2, hl2, dinv, bg2, g2, b2, wg3t)[0]


def _tc4_body(a3_ref, hl3_ref, dinv_ref, bg3_ref, wvt_ref, bv_ref, wot_ref,
              bo_ref, wc1t_ref, bc1_ref, wc2t_ref, bc2_ref, wc3t_ref,
              bc3_ref, out_ref):
    dinv = dinv_ref[...]
    agg = a3_ref[0] + a3_ref[1] + hl3_ref[...] * dinv
    h3 = jnp.maximum(agg * dinv + bg3_ref[...], 0.0)
    vv = jnp.dot(h3, wvt_ref[...], preferred_element_type=jnp.float32) \
        + bv_ref[...]
    att = jnp.dot(vv, wot_ref[...], preferred_element_type=jnp.float32) \
        + bo_ref[...]
    p = jnp.maximum(
        jnp.dot(att, wc1t_ref[...], preferred_element_type=jnp.float32)
        + bc1_ref[...], 0.0)
    p = jnp.maximum(
        jnp.dot(p, wc2t_ref[...], preferred_element_type=jnp.float32)
        + bc2_ref[...], 0.0)
    out_ref[...] = jnp.dot(p, wc3t_ref[...],
                           preferred_element_type=jnp.float32) + bc3_ref[...]


def _tc4(a3, hl3, dinv, bg3, wvt, bv, wot, bo, wc1t, bc1, wc2t, bc2, wc3t,
         bc3):
    return pl.pallas_call(
        _tc4_body,
        grid=(GRID,),
        in_specs=[
            _rows((2, RB, 16)),
            _rows((RB, 16)),
            _rows((RB, 1)),
            _full((1, 16)),
            _full((16, 16)),
            _full((1, 16)),
            _full((16, 16)),
            _full((1, 16)),
            _full((16, 8)),
            _full((1, 8)),
            _full((8, 32)),
            _full((1, 32)),
            _full((32, 1)),
            _full((1, 1)),
        ],
        out_specs=[_rows((RB, 1))],
        out_shape=[jax.ShapeDtypeStruct((NPAD, 1), jnp.float32)],
    )(a3, hl3, dinv, bg3, wvt, bv, wot, bo, wc1t, bc1, wc2t, bc2, wc3t,
      bc3)[0]


def kernel(x, edge_index, W_in, b_in, ln1_g, ln1_b, Wg1, bg1, Wg2, bg2,
           ln2_g, ln2_b, Wg3, bg3, Wqkv, bqkv, Wo, bo, Wc1, bc1, Wc2, bc2,
           Wc3, bc3):
    e3 = jnp.transpose(edge_index.reshape(2, EROWS, LANES), (1, 0, 2))

    xt = x.T
    wint = W_in.T

    degp = _deg_call(e3)
    deg = degp[0, :, 0] + degp[1, :, 0] + 1.0
    dinv1 = lax.rsqrt(deg)
    dinv = jnp.pad(dinv1, (0, NPAD - NACC),
                   constant_values=1.0).reshape(NPAD, 1)

    h, hl1 = _tc1(xt, wint, b_in.reshape(1, 64), ln1_g.reshape(1, 64),
                  ln1_b.reshape(1, 64), Wg1.T)

    hs1 = hl1 * dinv
    a1 = _agg64(hs1[:, :32], hs1[:, 32:], e3)

    hl2 = _tc2(h, a1, hl1, dinv, bg1.reshape(1, 64), Wg2.T)

    a2 = _agg32(hl2 * dinv, e3)

    hl3 = _tc3(a2, hl2, dinv, bg2.reshape(1, 32), ln2_g.reshape(1, 32),
               ln2_b.reshape(1, 32), Wg3.T)

    a3 = _agg16(hl3 * dinv, e3)

    wv = Wqkv[32:48]
    bv = bqkv[32:48]
    out = _tc4(a3, hl3, dinv, bg3.reshape(1, 16), wv.T, bv.reshape(1, 16),
               Wo.T, bo.reshape(1, 16), Wc1.T, bc1.reshape(1, 8), Wc2.T,
               bc2.reshape(1, 32), Wc3.T, bc3.reshape(1, 1))
    return out[:N, 0]


# speedup vs baseline: 2.0311x; 1.1921x over previous
import functools

import jax
import jax.numpy as jnp
from jax import lax
from jax.experimental import pallas as pl
from jax.experimental.pallas import tpu as pltpu
from jax.experimental.pallas import tpu_sc as plsc

N = 50000
E = 800000
LANES = 128
EROWS = E // LANES
NACC = 50048
CH = 40
ZROWS = 92
NPAD = 51200
RB = 2048
GRID = NPAD // RB


def _fill2d(ref, rows, width, value):
    v = jnp.full((16,), value, dtype=jnp.float32)

    def body(i, _):
        for f in range(width // 16):
            ref[i, pl.ds(f * 16, 16)] = v
        return 0

    lax.fori_loop(0, rows, body, 0)


def _sc_mesh():
    return plsc.VectorSubcoreMesh(core_axis_name="c", subcore_axis_name="s")


_SC_PARAMS = pltpu.CompilerParams(use_tc_tiling_on_sc=False)


def _make_deg_kernel():
    @functools.partial(
        pl.kernel,
        out_type=jax.ShapeDtypeStruct((2, NACC, 16), jnp.float32),
        mesh=_sc_mesh(),
        compiler_params=_SC_PARAMS,
        scratch_types=[
            pltpu.VMEM((CH, 2, LANES), jnp.int32),
            pltpu.VMEM((LANES, 16), jnp.float32),
            pltpu.VMEM((ZROWS, 16), jnp.float32),
            pltpu.VMEM_SHARED((NACC, 16), jnp.float32),
        ],
    )
    def k(e3, out, sd, ones_v, zbuf, acc):
        c = lax.axis_index("c")
        s = lax.axis_index("s")
        _fill2d(ones_v, LANES, 16, 1.0)
        _fill2d(zbuf, ZROWS, 16, 0.0)
        zbase = s * (NACC // 16)
        for z in range(34):
            pltpu.sync_copy(zbuf, acc.at[pl.ds(zbase + z * ZROWS, ZROWS)])
        plsc.subcore_barrier()

        wid = c * 16 + s
        row_base = wid * 195 + jnp.minimum(wid, 10)

        def run_rows(rb, nrows):
            pltpu.sync_copy(e3.at[pl.ds(rb, nrows)], sd.at[pl.ds(0, nrows)])

            def inner(j, _):
                pltpu.sync_copy(ones_v, acc.at[sd.at[j, 1]], add=True)
                return 0

            lax.fori_loop(0, nrows, inner, 0)

        def outer(ob, _):
            run_rows(row_base + ob * CH, CH)
            return 0

        lax.fori_loop(0, 4, outer, 0)
        run_rows(row_base + 4 * CH, 35)

        @pl.when(wid < 10)
        def _():
            run_rows(row_base + 195, 1)

        plsc.subcore_barrier()
        nout = NACC // 16
        pltpu.sync_copy(acc.at[pl.ds(s * nout, nout)],
                        out.at[c, pl.ds(s * nout, nout)])

    return k


def _make_agg_kernel(f2, colsplit):
    n_in = 2 if colsplit else 1
    base_rows, nfull, tail = (390, 9, 30) if colsplit else (195, 4, 35)

    def body(*refs):
        tables = refs[:n_in]
        e3 = refs[n_in]
        out = refs[n_in + 1]
        sd = refs[n_in + 2]
        bufs = refs[n_in + 3:n_in + 7]
        zbuf, acc = refs[n_in + 7], refs[n_in + 8]
        gsems = refs[n_in + 9:n_in + 13]
        ssems = refs[n_in + 13:n_in + 17]
        c = lax.axis_index("c")
        s = lax.axis_index("s")
        _fill2d(zbuf, ZROWS, f2, 0.0)
        zbase = s * (NACC // 16)
        for z in range(34):
            pltpu.sync_copy(zbuf, acc.at[pl.ds(zbase + z * ZROWS, ZROWS)])
        plsc.subcore_barrier()

        if colsplit:
            row_base = s * base_rows + jnp.minimum(s, 10)
            extra = s < 10
        else:
            wid = c * 16 + s
            row_base = wid * base_rows + jnp.minimum(wid, 10)
            extra = wid < 10

        def gather(j, k):
            if colsplit:
                @pl.when(c == 0)
                def _():
                    pltpu.async_copy(tables[0].at[sd.at[j, 0]], bufs[k],
                                     gsems[k])

                @pl.when(c == 1)
                def _():
                    pltpu.async_copy(tables[1].at[sd.at[j, 0]], bufs[k],
                                     gsems[k])
            else:
                pltpu.async_copy(tables[0].at[sd.at[j, 0]], bufs[k],
                                 gsems[k])

        def gwait(j, k):
            pltpu.make_async_copy(tables[0].at[sd.at[j, 0]], bufs[k],
                                  gsems[k]).wait()

        def scat(j, k):
            pltpu.async_copy(bufs[k], acc.at[sd.at[j, 1]], ssems[k],
                             add=True)

        def swait(j, k):
            pltpu.make_async_copy(bufs[k], acc.at[sd.at[j, 1]],
                                  ssems[k]).wait()

        def run_rows(rb, nrows):
            pltpu.sync_copy(e3.at[pl.ds(rb, nrows)], sd.at[pl.ds(0, nrows)])
            nq = nrows // 4
            rem = nrows - 4 * nq
            if nq == 0:
                for j in range(nrows):
                    gather(j, j)
                    gwait(j, j)
                    pltpu.sync_copy(bufs[j], acc.at[sd.at[j, 1]], add=True)
                return
            for u in range(4):
                gather(u, u)

            def quad(j4, _):
                j0 = 4 * j4
                for u in range(4):
                    gwait(j0 + u, u)
                    scat(j0 + u, u)

                @pl.when(j4 + 1 < nq)
                def _():
                    for u in range(4):
                        swait(j0 + u, u)
                        gather(j0 + 4 + u, u)
                return 0

            lax.fori_loop(0, nq, quad, 0)
            for u in range(4):
                swait(4 * nq - 4 + u, u)
            for u in range(rem):
                j = 4 * nq + u
                gather(j, u)
                gwait(j, u)
                pltpu.sync_copy(bufs[u], acc.at[sd.at[j, 1]], add=True)

        def outer(ob, _):
            run_rows(row_base + ob * CH, CH)
            return 0

        lax.fori_loop(0, nfull, outer, 0)
        run_rows(row_base + nfull * CH, tail)

        @pl.when(extra)
        def _():
            run_rows(row_base + base_rows, 1)

        plsc.subcore_barrier()
        nout = NACC // 16
        pltpu.sync_copy(acc.at[pl.ds(s * nout, nout)],
                        out.at[c, pl.ds(s * nout, nout)])

    return functools.partial(
        pl.kernel,
        out_type=jax.ShapeDtypeStruct((2, NACC, f2), jnp.float32),
        mesh=_sc_mesh(),
        compiler_params=_SC_PARAMS,
        scratch_types=[
            pltpu.VMEM((CH, 2, LANES), jnp.int32),
            pltpu.VMEM((LANES, f2), jnp.float32),
            pltpu.VMEM((LANES, f2), jnp.float32),
            pltpu.VMEM((LANES, f2), jnp.float32),
            pltpu.VMEM((LANES, f2), jnp.float32),
            pltpu.VMEM((ZROWS, f2), jnp.float32),
            pltpu.VMEM_SHARED((NACC, f2), jnp.float32),
        ] + [pltpu.SemaphoreType.DMA] * 8,
    )(body)


_deg_call = _make_deg_kernel()
_agg64 = _make_agg_kernel(32, colsplit=True)
_agg32 = _make_agg_kernel(32, colsplit=False)
_agg16 = _make_agg_kernel(16, colsplit=False)


def _full(shape):
    return pl.BlockSpec(shape, lambda i: (0,) * len(shape))


def _rows(shape):
    nd = len(shape)
    if nd == 2:
        return pl.BlockSpec(shape, lambda i: (i, 0))
    return pl.BlockSpec(shape, lambda i: (0, i, 0))


def _tc1_body(x_ref, wint_ref, bin_ref, g1_ref, b1_ref, wg1t_ref,
              h_ref, hl1_ref):
    xb = x_ref[...]
    h0 = jnp.maximum(
        lax.dot_general(xb, wint_ref[...], (((0,), (0,)), ((), ())),
                        preferred_element_type=jnp.float32)
        + bin_ref[...], 0.0)
    m = jnp.mean(h0, axis=-1, keepdims=True)
    v = jnp.mean((h0 - m) ** 2, axis=-1, keepdims=True)
    hb = (h0 - m) / jnp.sqrt(v + 1e-5) * g1_ref[...] + b1_ref[...]
    h_ref[...] = hb
    hl1_ref[...] = jnp.dot(hb, wg1t_ref[...],
                           preferred_element_type=jnp.float32)


def _tc1(xp, wint, bin_, g1, b1, wg1t):
    return pl.pallas_call(
        _tc1_body,
        grid=(GRID,),
        in_specs=[
            pl.BlockSpec((164, RB), lambda i: (0, i)),
            _full((164, 64)),
            _full((1, 64)),
            _full((1, 64)),
            _full((1, 64)),
            _full((64, 64)),
        ],
        out_specs=[_rows((RB, 64)), _rows((RB, 64))],
        out_shape=[
            jax.ShapeDtypeStruct((NPAD, 64), jnp.float32),
            jax.ShapeDtypeStruct((NPAD, 64), jnp.float32),
        ],
    )(xp, wint, bin_, g1, b1, wg1t)


def _tc2_body(h_ref, a1_ref, hl1_ref, dinv_ref, bg1_ref, wg2t_ref, hl2_ref):
    dinv = dinv_ref[...]
    hs1 = hl1_ref[...] * dinv
    left = a1_ref[0] + hs1[:, :32]
    right = a1_ref[1] + hs1[:, 32:]
    agg = jnp.concatenate([left, right], axis=1)
    t = jnp.maximum(agg * dinv + bg1_ref[...], 0.0)
    h1 = t + h_ref[...]
    hl2_ref[...] = jnp.dot(h1, wg2t_ref[...],
                           preferred_element_type=jnp.float32)


def _tc2(h, a1, hl1, dinv, bg1, wg2t):
    return pl.pallas_call(
        _tc2_body,
        grid=(GRID,),
        in_specs=[
            _rows((RB, 64)),
            _rows((2, RB, 32)),
            _rows((RB, 64)),
            _rows((RB, 1)),
            _full((1, 64)),
            _full((64, 32)),
        ],
        out_specs=[_rows((RB, 32))],
        out_shape=[jax.ShapeDtypeStruct((NPAD, 32), jnp.float32)],
    )(h, a1, hl1, dinv, bg1, wg2t)[0]


def _tc3_body(a2_ref, hl2_ref, dinv_ref, bg2_ref, g2_ref, b2_ref, wg3t_ref,
              hl3_ref):
    dinv = dinv_ref[...]
    agg = a2_ref[0] + a2_ref[1] + hl2_ref[...] * dinv
    t = jnp.maximum(agg * dinv + bg2_ref[...], 0.0)
    m = jnp.mean(t, axis=-1, keepdims=True)
    v = jnp.mean((t - m) ** 2, axis=-1, keepdims=True)
    h2 = (t - m) / jnp.sqrt(v + 1e-5) * g2_ref[...] + b2_ref[...]
    hl3_ref[...] = jnp.dot(h2, wg3t_ref[...],
                           preferred_element_type=jnp.float32)


def _tc3(a2, hl2, dinv, bg2, g2, b2, wg3t):
    return pl.pallas_call(
        _tc3_body,
        grid=(GRID,),
        in_specs=[
            _rows((2, RB, 32)),
            _rows((RB, 32)),
            _rows((RB, 1)),
            _full((1, 32)),
            _full((1, 32)),
            _full((1, 32)),
            _full((32, 16)),
        ],
        out_specs=[_rows((RB, 16))],
        out_shape=[jax.ShapeDtypeStruct((NPAD, 16), jnp.float32)],
    )(a2, hl2, dinv, bg2, g2, b2, wg3t)[0]


def _tc4_body(a3_ref, hl3_ref, dinv_ref, bg3_ref, wvt_ref, bv_ref, wot_ref,
              bo_ref, wc1t_ref, bc1_ref, wc2t_ref, bc2_ref, wc3t_ref,
              bc3_ref, out_ref):
    dinv = dinv_ref[...]
    agg = a3_ref[0] + a3_ref[1] + hl3_ref[...] * dinv
    h3 = jnp.maximum(agg * dinv + bg3_ref[...], 0.0)
    vv = jnp.dot(h3, wvt_ref[...], preferred_element_type=jnp.float32) \
        + bv_ref[...]
    att = jnp.dot(vv, wot_ref[...], preferred_element_type=jnp.float32) \
        + bo_ref[...]
    p = jnp.maximum(
        jnp.dot(att, wc1t_ref[...], preferred_element_type=jnp.float32)
        + bc1_ref[...], 0.0)
    p = jnp.maximum(
        jnp.dot(p, wc2t_ref[...], preferred_element_type=jnp.float32)
        + bc2_ref[...], 0.0)
    out_ref[...] = jnp.dot(p, wc3t_ref[...],
                           preferred_element_type=jnp.float32) + bc3_ref[...]


def _tc4(a3, hl3, dinv, bg3, wvt, bv, wot, bo, wc1t, bc1, wc2t, bc2, wc3t,
         bc3):
    return pl.pallas_call(
        _tc4_body,
        grid=(GRID,),
        in_specs=[
            _rows((2, RB, 16)),
            _rows((RB, 16)),
            _rows((RB, 1)),
            _full((1, 16)),
            _full((16, 16)),
            _full((1, 16)),
            _full((16, 16)),
            _full((1, 16)),
            _full((16, 8)),
            _full((1, 8)),
            _full((8, 32)),
            _full((1, 32)),
            _full((32, 1)),
            _full((1, 1)),
        ],
        out_specs=[_rows((RB, 1))],
        out_shape=[jax.ShapeDtypeStruct((NPAD, 1), jnp.float32)],
    )(a3, hl3, dinv, bg3, wvt, bv, wot, bo, wc1t, bc1, wc2t, bc2, wc3t,
      bc3)[0]


def kernel(x, edge_index, W_in, b_in, ln1_g, ln1_b, Wg1, bg1, Wg2, bg2,
           ln2_g, ln2_b, Wg3, bg3, Wqkv, bqkv, Wo, bo, Wc1, bc1, Wc2, bc2,
           Wc3, bc3):
    e3 = jnp.transpose(edge_index.reshape(2, EROWS, LANES), (1, 0, 2))

    xt = x.T
    wint = W_in.T

    degp = _deg_call(e3)
    deg = degp[0, :, 0] + degp[1, :, 0] + 1.0
    dinv1 = lax.rsqrt(deg)
    dinv = jnp.pad(dinv1, (0, NPAD - NACC),
                   constant_values=1.0).reshape(NPAD, 1)

    h, hl1 = _tc1(xt, wint, b_in.reshape(1, 64), ln1_g.reshape(1, 64),
                  ln1_b.reshape(1, 64), Wg1.T)

    hs1 = hl1 * dinv
    a1 = _agg64(hs1[:, :32], hs1[:, 32:], e3)

    hl2 = _tc2(h, a1, hl1, dinv, bg1.reshape(1, 64), Wg2.T)

    a2 = _agg32(hl2 * dinv, e3)

    hl3 = _tc3(a2, hl2, dinv, bg2.reshape(1, 32), ln2_g.reshape(1, 32),
               ln2_b.reshape(1, 32), Wg3.T)

    a3 = _agg16(hl3 * dinv, e3)

    wv = Wqkv[32:48]
    bv = bqkv[32:48]
    out = _tc4(a3, hl3, dinv, bg3.reshape(1, 16), wv.T, bv.reshape(1, 16),
               Wo.T, bo.reshape(1, 16), Wc1.T, bc1.reshape(1, 8), Wc2.T,
               bc2.reshape(1, 32), Wc3.T, bc3.reshape(1, 1))
    return out[:N, 0]
